# baseline (speedup 1.0000x reference)
import dataclasses
import functools

import jax
import jax.numpy as jnp
from jax import lax
from jax.experimental import pallas as pl
from jax.experimental.pallas import tpu as pltpu
from jax.experimental.pallas import tpu_sc as plsc

N_NODES = 10000
N_EDGES = 160000
D_FEAT = 128
D_EDGE = 16
HEADS = 8
HID = 32
HC = HEADS * HID
HALF = HC // 2
ACC_W = 144

N_TILES = 16
EDGE_BLK = 64
EDGES_PER_TILE = N_EDGES // N_TILES
NPAD = 10240
ROWS_PER_TILE = NPAD // N_TILES

ROW_BLK = 1024
NB = NPAD // ROW_BLK
E_BLK = 2000
NEB = N_EDGES // E_BLK

_f32 = jnp.float32


def _head_masks(half_heads):
    r = lax.broadcasted_iota(jnp.int32, (HALF, half_heads), 0) // HID
    c = lax.broadcasted_iota(jnp.int32, (HALF, half_heads), 1)
    m = (r == c).astype(_f32)
    rt = lax.broadcasted_iota(jnp.int32, (half_heads, HALF), 1) // HID
    ct = lax.broadcasted_iota(jnp.int32, (half_heads, HALF), 0)
    mt = (rt == ct).astype(_f32)
    return m, mt


def _leaky(x):
    return jnp.maximum(x, 0.2 * x)


def _mean_ea_body(ea_ref, o_ref):
    @pl.when(pl.program_id(0) == 0)
    def _():
        o_ref[...] = jnp.zeros_like(o_ref)

    o_ref[...] += jnp.sum(ea_ref[...], axis=0, keepdims=True) * (1.0 / N_EDGES)


def _mean_ea(edge_attr):
    return pl.pallas_call(
        _mean_ea_body,
        grid=(20,),
        in_specs=[pl.BlockSpec((N_EDGES // 20, D_EDGE), lambda i: (i, 0))],
        out_specs=pl.BlockSpec((1, D_EDGE), lambda i: (0, 0)),
        out_shape=jax.ShapeDtypeStruct((1, D_EDGE), _f32),
    )(edge_attr)


def _eproj_body(ea_ref, we_ref, o_ref):
    o_ref[...] = jnp.dot(ea_ref[...], we_ref[...], preferred_element_type=_f32)


def _eproj(edge_attr, We):
    return pl.pallas_call(
        _eproj_body,
        grid=(2, NEB),
        in_specs=[
            pl.BlockSpec((E_BLK, D_EDGE), lambda h, i: (i, 0)),
            pl.BlockSpec((D_EDGE, HALF), lambda h, i: (0, h)),
        ],
        out_specs=pl.BlockSpec((E_BLK, HALF), lambda h, i: (h * NEB + i, 0)),
        out_shape=jax.ShapeDtypeStruct((2 * N_EDGES, HALF), _f32),
    )(edge_attr, We)


def _dense_half(h, wl, bl, wr, br, eloop, attf):
    xl = jnp.dot(h, wl, preferred_element_type=_f32) + bl
    xr = jnp.dot(h, wr, preferred_element_type=_f32) + br
    m = _leaky(xl + xr + eloop)
    t = m * attf
    cm, cmt = _head_masks(HEADS // 2)
    alpha = jnp.dot(t, cm, preferred_element_type=_f32)
    ex = jnp.exp(alpha)
    exrep = jnp.dot(ex, cmt, preferred_element_type=_f32)
    acc = jnp.concatenate(
        [xl * exrep, ex, jnp.zeros((h.shape[0], ACC_W - HALF - 4), _f32)], axis=1
    )
    return xl, xr, acc


def _normalize(acc_lo, acc_hi, mlo, mhi, dlo, dhi, bias):
    main = jnp.concatenate(
        [acc_lo[:, :HALF] + mlo, acc_hi[:, :HALF] + mhi], axis=1)
    dl = acc_lo[:, HALF : HALF + 4] + jnp.sum(dlo[0], axis=0)
    dh = acc_hi[:, HALF : HALF + 4] + jnp.sum(dhi[0], axis=0)
    den = jnp.concatenate([dl, dh], axis=1)
    r = lax.broadcasted_iota(jnp.int32, (HEADS, HC), 1) // HID
    c = lax.broadcasted_iota(jnp.int32, (HEADS, HC), 0)
    cmt = (r == c).astype(_f32)
    denrep = jnp.dot(den, cmt, preferred_element_type=_f32)
    return _leaky(main / (denrep + 1e-16) + bias)


def _prep0_body(x_ref, wl_ref, bl_ref, wr_ref, br_ref, we_ref, attf_ref,
                mea_ref, xl_ref, xr_ref, acc_ref):
    eloop = jnp.dot(mea_ref[...], we_ref[...], preferred_element_type=_f32)
    xl, xr, acc = _dense_half(
        x_ref[...], wl_ref[...], bl_ref[...], wr_ref[...], br_ref[...],
        eloop, attf_ref[...])
    xl_ref[...] = xl
    xr_ref[...] = xr
    acc_ref[...] = acc


def _prep0(x, conv, attf, mean_ea):
    wspec = lambda d: pl.BlockSpec((d, HALF), lambda h, i: (0, h))
    return pl.pallas_call(
        _prep0_body,
        grid=(2, NB),
        in_specs=[
            pl.BlockSpec((ROW_BLK, D_FEAT), lambda h, i: (i, 0)),
            wspec(D_FEAT),
            pl.BlockSpec((1, HALF), lambda h, i: (0, h)),
            wspec(D_FEAT),
            pl.BlockSpec((1, HALF), lambda h, i: (0, h)),
            wspec(D_EDGE),
            pl.BlockSpec((1, HALF), lambda h, i: (0, h)),
            pl.BlockSpec((1, D_EDGE), lambda h, i: (0, 0)),
        ],
        out_specs=[
            pl.BlockSpec((ROW_BLK, HALF), lambda h, i: (h * NB + i, 0)),
            pl.BlockSpec((ROW_BLK, HALF), lambda h, i: (h * NB + i, 0)),
            pl.BlockSpec((ROW_BLK, ACC_W), lambda h, i: (h * NB + i, 0)),
        ],
        out_shape=[
            jax.ShapeDtypeStruct((2 * NPAD, HALF), _f32),
            jax.ShapeDtypeStruct((2 * NPAD, HALF), _f32),
            jax.ShapeDtypeStruct((2 * NPAD, ACC_W), _f32),
        ],
    )(x, conv["Wl"], conv["bl"].reshape(1, HC), conv["Wr"],
      conv["br"].reshape(1, HC), conv["We"], attf, mean_ea)


def _prepn_body(alo_ref, ahi_ref, mlo_ref, mhi_ref, dlo_ref, dhi_ref,
                pbias_ref, wl_ref, bl_ref, wr_ref, br_ref,
                we_ref, attf_ref, mea_ref, xl_ref, xr_ref, acc_ref):
    h = _normalize(alo_ref[...], ahi_ref[...], mlo_ref[...], mhi_ref[...],
                   dlo_ref[...], dhi_ref[...], pbias_ref[...])
    eloop = jnp.dot(mea_ref[...], we_ref[...], preferred_element_type=_f32)
    xl, xr, acc = _dense_half(
        h, wl_ref[...], bl_ref[...], wr_ref[...], br_ref[...],
        eloop, attf_ref[...])
    xl_ref[...] = xl
    xr_ref[...] = xr
    acc_ref[...] = acc


def _acc_in_specs():
    return [
        pl.BlockSpec((ROW_BLK, ACC_W), lambda h, i: (i, 0)),
        pl.BlockSpec((ROW_BLK, ACC_W), lambda h, i: (NB + i, 0)),
        pl.BlockSpec((ROW_BLK, HALF), lambda h, i: (i, 0)),
        pl.BlockSpec((ROW_BLK, HALF), lambda h, i: (NB + i, 0)),
        pl.BlockSpec((1, N_TILES, ROW_BLK, 4), lambda h, i: (0, 0, i, 0)),
        pl.BlockSpec((1, N_TILES, ROW_BLK, 4), lambda h, i: (1, 0, i, 0)),
    ]


def _prepn(acc_self, main_sc, den_sc, prev_bias, conv, attf, mean_ea):
    wspec = lambda d: pl.BlockSpec((d, HALF), lambda h, i: (0, h))
    return pl.pallas_call(
        _prepn_body,
        grid=(2, NB),
        in_specs=_acc_in_specs() + [
            pl.BlockSpec((1, HC), lambda h, i: (0, 0)),
            wspec(HC),
            pl.BlockSpec((1, HALF), lambda h, i: (0, h)),
            wspec(HC),
            pl.BlockSpec((1, HALF), lambda h, i: (0, h)),
            wspec(D_EDGE),
            pl.BlockSpec((1, HALF), lambda h, i: (0, h)),
            pl.BlockSpec((1, D_EDGE), lambda h, i: (0, 0)),
        ],
        out_specs=[
            pl.BlockSpec((ROW_BLK, HALF), lambda h, i: (h * NB + i, 0)),
            pl.BlockSpec((ROW_BLK, HALF), lambda h, i: (h * NB + i, 0)),
            pl.BlockSpec((ROW_BLK, ACC_W), lambda h, i: (h * NB + i, 0)),
        ],
        out_shape=[
            jax.ShapeDtypeStruct((2 * NPAD, HALF), _f32),
            jax.ShapeDtypeStruct((2 * NPAD, HALF), _f32),
            jax.ShapeDtypeStruct((2 * NPAD, ACC_W), _f32),
        ],
    )(acc_self, acc_self, main_sc, main_sc, den_sc, den_sc,
      prev_bias.reshape(1, HC), conv["Wl"], conv["bl"].reshape(1, HC),
      conv["Wr"], conv["br"].reshape(1, HC), conv["We"], attf, mean_ea)


def _mlp_body(alo_ref, ahi_ref, mlo_ref, mhi_ref, dlo_ref, dhi_ref,
              pbias_ref, w1, b1, w2, b2, w3, b3, w4, b4, o_ref):
    h = _normalize(alo_ref[...], ahi_ref[...], mlo_ref[...], mhi_ref[...],
                   dlo_ref[...], dhi_ref[...], pbias_ref[...])
    h = jnp.maximum(jnp.dot(h, w1[...], preferred_element_type=_f32) + b1[...], 0.0)
    h = jnp.maximum(jnp.dot(h, w2[...], preferred_element_type=_f32) + b2[...], 0.0)
    h = jnp.maximum(jnp.dot(h, w3[...], preferred_element_type=_f32) + b3[...], 0.0)
    o_ref[...] = jnp.dot(h, w4[...], preferred_element_type=_f32) + b4[...]


def _mlp(acc_self, main_sc, den_sc, prev_bias, lins):
    full = lambda a, b: pl.BlockSpec((a, b), lambda h, i: (0, 0))
    l1, l2, l3, l4 = lins
    return pl.pallas_call(
        _mlp_body,
        grid=(1, NB),
        in_specs=_acc_in_specs() + [
            full(1, HC),
            full(HC, 16), full(1, 16),
            full(16, 16), full(1, 16),
            full(16, 16), full(1, 16),
            full(16, D_FEAT), full(1, D_FEAT),
        ],
        out_specs=pl.BlockSpec((ROW_BLK, D_FEAT), lambda h, i: (i, 0)),
        out_shape=jax.ShapeDtypeStruct((N_NODES, D_FEAT), _f32),
    )(acc_self, acc_self, main_sc, main_sc, den_sc, den_sc,
      prev_bias.reshape(1, HC),
      l1["W"], l1["b"].reshape(1, 16),
      l2["W"], l2["b"].reshape(1, 16),
      l3["W"], l3["b"].reshape(1, 16),
      l4["W"], l4["b"].reshape(1, D_FEAT))


NHALF = 5120
ACC_ROWS = NHALF + 8
CAP = 168 * EDGE_BLK
MROWS = NHALF // N_TILES

_GDN = lax.GatherDimensionNumbers(
    offset_dims=(), collapsed_slice_dims=(0,), start_index_map=(0,))


def _vperm(v, idx):
    return lax.gather(v, idx.reshape(16, 1), _GDN, (1,),
                      mode=lax.GatherScatterMode.PROMISE_IN_BOUNDS)


def _rep4(v, g):
    lane = lax.iota(jnp.int32, 16)
    idx = (lane // 4 + 4 * g).reshape(16, 1)
    return lax.gather(v, idx, _GDN, (1,),
                      mode=lax.GatherScatterMode.PROMISE_IN_BOUNDS)


def _sc_compiler_params():
    cp = pltpu.CompilerParams()
    if "needs_layout_passes" in pltpu.CompilerParams.__dataclass_fields__:
        cp = dataclasses.replace(cp, needs_layout_passes=False)
    return cp


def _mesh():
    return plsc.VectorSubcoreMesh(core_axis_name="c", subcore_axis_name="s")


def _partition_body(src_hbm, dst_hbm, srcl_hbm, dstl_hbm, eidl_hbm, cnt_hbm,
                    sstage, dstage, srcout, dstout, eidout, cntv):
    c = lax.axis_index("c")
    s = lax.axis_index("s")
    base = s * EDGES_PER_TILE
    lane = lax.iota(jnp.int32, 16)
    zero_i = jnp.zeros((16,), jnp.int32)

    pltpu.sync_copy(src_hbm.at[pl.ds(base, EDGES_PER_TILE)], sstage)
    pltpu.sync_copy(dst_hbm.at[pl.ds(base, EDGES_PER_TILE)], dstage)

    @pl.loop(0, CAP, step=16)
    def _fill(j):
        for q in range(2):
            srcout[pl.ds(q * CAP + j, 16)] = zero_i
            dstout[pl.ds(q * CAP + j, 16)] = jnp.full(
                (16,), q * NHALF + NHALF, jnp.int32)
            eidout[pl.ds(q * CAP + j, 16)] = zero_i

    def _step(t, offs):
        off0, off1 = offs
        j = t * 16
        sv = sstage[pl.ds(j, 16)]
        dv = dstage[pl.ds(j, 16)]
        ev = base + j + lane
        m0 = dv < NHALF
        m1 = jnp.logical_not(m0)
        plsc.store_compressed(srcout.at[pl.ds(off0, 16)], sv, mask=m0)
        plsc.store_compressed(dstout.at[pl.ds(off0, 16)], dv, mask=m0)
        plsc.store_compressed(eidout.at[pl.ds(off0, 16)], ev, mask=m0)
        plsc.store_compressed(srcout.at[pl.ds(CAP + off1, 16)], sv, mask=m1)
        plsc.store_compressed(dstout.at[pl.ds(CAP + off1, 16)], dv, mask=m1)
        plsc.store_compressed(eidout.at[pl.ds(CAP + off1, 16)], ev, mask=m1)
        n0 = jnp.sum(m0.astype(jnp.int32))
        return off0 + n0, off1 + (16 - n0)

    off0, off1 = lax.fori_loop(0, EDGES_PER_TILE // 16, _step,
                               (jnp.int32(0), jnp.int32(0)))
    nblk0 = ((off0 + (EDGE_BLK - 1)) // EDGE_BLK + 1) // 2 * 2
    nblk1 = ((off1 + (EDGE_BLK - 1)) // EDGE_BLK + 1) // 2 * 2
    cntv[...] = (jnp.where(lane == 0, nblk0, 0)
                 + jnp.where(lane == 1, nblk1, 0)
                 + jnp.where(lane == 2, off0, 0)
                 + jnp.where(lane == 3, off1, 0))
    w = c * N_TILES + s
    pltpu.sync_copy(cntv, cnt_hbm.at[pl.ds(w * 16, 16)])
    for q in range(2):
        lb = (w * 2 + q) * CAP
        pltpu.sync_copy(srcout.at[pl.ds(q * CAP, CAP)],
                        srcl_hbm.at[pl.ds(lb, CAP)])
        pltpu.sync_copy(dstout.at[pl.ds(q * CAP, CAP)],
                        dstl_hbm.at[pl.ds(lb, CAP)])
        pltpu.sync_copy(eidout.at[pl.ds(q * CAP, CAP)],
                        eidl_hbm.at[pl.ds(lb, CAP)])


def _partition(src, dst):
    call = pl.kernel(
        _partition_body,
        out_type=[
            jax.ShapeDtypeStruct((64 * CAP,), jnp.int32),
            jax.ShapeDtypeStruct((64 * CAP,), jnp.int32),
            jax.ShapeDtypeStruct((64 * CAP,), jnp.int32),
            jax.ShapeDtypeStruct((512,), jnp.int32),
        ],
        mesh=_mesh(),
        compiler_params=_sc_compiler_params(),
        scratch_types=[
            pltpu.VMEM((EDGES_PER_TILE,), jnp.int32),
            pltpu.VMEM((EDGES_PER_TILE,), jnp.int32),
            pltpu.VMEM((2 * CAP,), jnp.int32),
            pltpu.VMEM((2 * CAP,), jnp.int32),
            pltpu.VMEM((2 * CAP,), jnp.int32),
            pltpu.VMEM((16,), jnp.int32),
        ],
    )
    return call(src, dst)


def _edge_sc_body(q, srcl_hbm, dstl_hbm, eidl_hbm, cnt_hbm, xl_hbm, xr_hbm,
                  e_hbm, attf_hbm, dep_hbm, out_main_hbm, out_den_hbm,
                  sstg, dstg, estg,
                  srcv0, dstv20, dstv30, eidv0,
                  srcv1, dstv21, dstv31, eidv1,
                  xjv0, xrv0, ev0, cv0,
                  xjv1, xrv1, ev1, cv1,
                  attv, denp, cnts, accsh,
                  sg0, sg1, sg2, sg3, sg4, sg5):
    c = lax.axis_index("c")
    s = lax.axis_index("s")
    w = c * N_TILES + s
    wq = w * 2 + q
    zero16 = jnp.zeros((16,), _f32)
    lane = lax.iota(jnp.int32, 16)

    pltpu.sync_copy(cnt_hbm.at[pl.ds(w * 16, 16)], cnts)
    nblk = cnts[...][q]

    @pl.loop(0, 4 * ACC_ROWS, step=16)
    def _zd(j):
        denp[pl.ds(j, 16)] = zero16

    @pl.loop(0, EDGE_BLK)
    def _zc(i):
        for k in range(8):
            cv0[i, pl.ds(16 * k, 16)] = zero16

    row0 = s * MROWS

    @pl.loop(0, MROWS, step=EDGE_BLK)
    def _zs(r):
        pltpu.sync_copy(cv0, accsh.at[pl.ds(row0 + r, EDGE_BLK)])

    @pl.when(s == N_TILES - 1)
    def _zt():
        pltpu.sync_copy(cv0.at[pl.ds(0, 8)], accsh.at[pl.ds(NHALF, 8)])

    pltpu.sync_copy(attf_hbm.at[pl.ds(c * HALF, HALF)], attv)
    plsc.subcore_barrier()

    att_regs = [attv[pl.ds(16 * k, 16)] for k in range(8)]
    coff_n = c * NPAD
    coff_e = c * N_EDGES
    qoff = q * NHALF
    lbase = wq * CAP

    bufs = [
        (srcv0, dstv20, dstv30, eidv0, xjv0, xrv0, ev0, cv0, sg0, sg1, sg2),
        (srcv1, dstv21, dstv31, eidv1, xjv1, xrv1, ev1, cv1, sg3, sg4, sg5),
    ]

    def stage(b):
        off = lbase + b * EDGE_BLK
        n4 = 4 * EDGE_BLK
        pltpu.sync_copy(srcl_hbm.at[pl.ds(off, n4)], sstg)
        pltpu.sync_copy(dstl_hbm.at[pl.ds(off, n4)], dstg)
        pltpu.sync_copy(eidl_hbm.at[pl.ds(off, n4)], estg)

    def issue(b, t):
        srcv, dstv2, dstv3, eidv, xjv, xrv, ev, cv = bufs[t][:8]
        g0, g1, g2 = bufs[t][8:11]

        @pl.when(b % 4 == 0)
        def _st():
            stage(b)

        so = (b % 4) * EDGE_BLK

        @pl.loop(0, EDGE_BLK, step=16)
        def _shift(j):
            dv = dstg[pl.ds(so + j, 16)]
            srcv[pl.ds(j, 16)] = sstg[pl.ds(so + j, 16)] + coff_n
            dstv2[pl.ds(j, 16)] = jnp.minimum(dv + coff_n, 2 * NPAD - 1)
            dstv3[pl.ds(j, 16)] = dv - qoff
            eidv[pl.ds(j, 16)] = estg[pl.ds(so + j, 16)] + coff_e

        pltpu.async_copy(xl_hbm.at[srcv], xjv, g0)
        pltpu.async_copy(xr_hbm.at[dstv2], xrv, g1)
        pltpu.async_copy(e_hbm.at[eidv], ev, g2)

    def wait_gathers(t):
        srcv, dstv2, dstv3, eidv, xjv, xrv, ev, cv = bufs[t][:8]
        g0, g1, g2 = bufs[t][8:11]
        pltpu.make_async_copy(xl_hbm.at[srcv], xjv, g0).wait()
        pltpu.make_async_copy(xr_hbm.at[dstv2], xrv, g1).wait()
        pltpu.make_async_copy(e_hbm.at[eidv], ev, g2).wait()

    def compute_scatter(t):
        srcv, dstv2, dstv3, eidv, xjv, xrv, ev, cv = bufs[t][:8]

        @pl.loop(0, EDGE_BLK, step=16)
        def _grp(j):
            dchunk = dstv3[pl.ds(j, 16)]
            for g in range(4):
                a_sc = []
                for e in range(4):
                    i = j + 4 * g + e
                    for h in range(4):
                        k0, k1 = 2 * h, 2 * h + 1
                        xj0 = xjv[i, pl.ds(16 * k0, 16)]
                        xj1 = xjv[i, pl.ds(16 * k1, 16)]
                        m0 = _leaky(xj0 + xrv[i, pl.ds(16 * k0, 16)]
                                    + ev[i, pl.ds(16 * k0, 16)])
                        m1 = _leaky(xj1 + xrv[i, pl.ds(16 * k1, 16)]
                                    + ev[i, pl.ds(16 * k1, 16)])
                        ph = m0 * att_regs[k0] + m1 * att_regs[k1]
                        a = jnp.sum(ph)
                        exb = jnp.exp(jnp.full((16,), a, _f32))
                        cv[i, pl.ds(16 * k0, 16)] = xj0 * exb
                        cv[i, pl.ds(16 * k1, 16)] = xj1 * exb
                        a_sc.append(a)
                av = zero16
                for t2, a in enumerate(a_sc):
                    av = av + jnp.where(lane == t2, a, 0.0)
                exv = jnp.exp(av)
                drep = _rep4(dchunk, g)
                didx = drep * 4 + (lane % 4)
                plsc.addupdate_scatter(denp, [didx], exv)

        pltpu.sync_copy(cv, accsh.at[dstv3], add=True)

    issue(0, 0)

    @pl.loop(0, nblk // 2)
    def _blk2(g):
        b = g * 2
        issue(b + 1, 1)
        wait_gathers(0)
        compute_scatter(0)
        issue(b + 2, 0)
        wait_gathers(1)
        compute_scatter(1)

    wait_gathers(0)

    plsc.subcore_barrier()
    pltpu.sync_copy(
        accsh.at[pl.ds(row0, MROWS)],
        out_main_hbm.at[pl.ds(c * NHALF + row0, MROWS)])
    pltpu.sync_copy(denp,
                    out_den_hbm.at[pl.ds(w * 4 * ACC_ROWS, 4 * ACC_ROWS)])


def _edge_layer_q(q, srcl, dstl, eidl, cnts, xl_tab, xr_tab, e_tab, attf_flat,
                  dep):
    call = pl.kernel(
        functools.partial(_edge_sc_body, q),
        out_type=[
            jax.ShapeDtypeStruct((2 * NHALF, HALF), _f32),
            jax.ShapeDtypeStruct((32 * 4 * ACC_ROWS,), _f32),
        ],
        mesh=_mesh(),
        compiler_params=_sc_compiler_params(),
        scratch_types=(
            [pltpu.VMEM((4 * EDGE_BLK,), jnp.int32) for _ in range(3)]
            + [pltpu.VMEM((EDGE_BLK,), jnp.int32) for _ in range(8)]
            + [pltpu.VMEM((EDGE_BLK, HALF), _f32) for _ in range(8)]
            + [
                pltpu.VMEM((HALF,), _f32),
                pltpu.VMEM((4 * ACC_ROWS,), _f32),
                pltpu.VMEM((16,), jnp.int32),
                pltpu.VMEM_SHARED((ACC_ROWS, HALF), _f32),
            ]
            + [pltpu.SemaphoreType.DMA for _ in range(6)]
        ),
    )
    return call(srcl, dstl, eidl, cnts, xl_tab, xr_tab, e_tab, attf_flat,
                dep)


def kernel(x, edge_index, edge_attr, params):
    src = edge_index[0]
    dst = edge_index[1]
    convs = params["convs"]
    lins = params["lins"]

    x = jnp.pad(x, ((0, NPAD - N_NODES), (0, 0)))
    mean_ea = _mean_ea(edge_attr)
    attfs = [c["att"].reshape(1, HC) for c in convs]
    attfs_flat = [a.reshape(HC) for a in attfs]

    srcl, dstl, eidl, cnts = _partition(src, dst)
    xl_tab, xr_tab, acc_self = _prep0(x, convs[0], attfs[0], mean_ea)
    main_sc = den_sc = None
    for l in range(3):
        e_tab = _eproj(edge_attr, convs[l]["We"])
        if l > 0:
            xl_tab, xr_tab, acc_self = _prepn(
                acc_self, main_sc, den_sc, convs[l - 1]["bias"], convs[l],
                attfs[l], mean_ea)
        mq, dq = [], []
        dep = cnts
        for qq in (0, 1):
            m, d = _edge_layer_q(qq, srcl, dstl, eidl, cnts, xl_tab, xr_tab,
                                 e_tab, attfs_flat[l], dep)
            mq.append(m)
            dq.append(d)
            dep = d
        main_sc = jnp.concatenate(
            [mq[0][:NHALF], mq[1][:NHALF], mq[0][NHALF:], mq[1][NHALF:]], 0)
        dr = [d.reshape(2, N_TILES, ACC_ROWS, 4)[:, :, :NHALF] for d in dq]
        den_sc = jnp.concatenate(dr, axis=2)
    return _mlp(acc_self, main_sc, den_sc, convs[2]["bias"], lins)
